# baseline (speedup 1.0000x reference)
import functools

import jax
import jax.numpy as jnp
from jax import lax
from jax.experimental import pallas as pl
from jax.experimental.pallas import tpu as pltpu
from jax.experimental.pallas import tpu_sc as plsc

VOCAB = 1000000
DIM = 100
BATCH = 16384
HIST = 200

_LANES = 16
_ROW_BLK = 2000


def _matvec_body(t_ref, w_ref, o_ref):
    o_ref[...] = (lax.dot_general(
        w_ref[...], t_ref[...],
        (((1,), (1,)), ((), ())),
        preferred_element_type=jnp.float32,
    ) * (1.0 / HIST))[None]


def _table_matvec(table, weights):
    wt = weights.reshape(1, DIM)
    return pl.pallas_call(
        _matvec_body,
        grid=(VOCAB // _ROW_BLK,),
        in_specs=[
            pl.BlockSpec((_ROW_BLK, DIM), lambda i: (i, 0)),
            pl.BlockSpec((1, DIM), lambda i: (0, 0)),
        ],
        out_specs=pl.BlockSpec((1, 1, _ROW_BLK), lambda i: (i, 0, 0)),
        out_shape=jax.ShapeDtypeStruct(
            (VOCAB // _ROW_BLK, 1, _ROW_BLK), jnp.float32),
    )(table, wt)


def _make_sc_gather_sum():
    nc, ns = 2, 16
    nw = nc * ns
    rows_w = BATCH // nw
    chunk_rows = 128
    n_chunks = rows_w // chunk_rows
    chunk_idx = chunk_rows * HIST

    mesh = plsc.VectorSubcoreMesh(core_axis_name="c", subcore_axis_name="s")
    n_full = HIST // _LANES
    tail_at = HIST - _LANES

    @functools.partial(
        pl.kernel,
        mesh=mesh,
        out_type=jax.ShapeDtypeStruct((BATCH * _LANES,), jnp.float32),
        scratch_types=[
            pltpu.VMEM((chunk_idx,), jnp.int32),
            pltpu.VMEM((chunk_idx,), jnp.float32),
            pltpu.VMEM((rows_w * _LANES,), jnp.float32),
            pltpu.SemaphoreType.DMA,
        ],
    )
    def sc_kernel(idx_hbm, s_hbm, out_hbm, idx_v, vals_v, part_v, sem):
        wid = lax.axis_index("s") * nc + lax.axis_index("c")
        row0 = wid * rows_w
        lane = lax.iota(jnp.int32, _LANES)
        tail_mask = jnp.where(
            lane >= (n_full * _LANES - tail_at),
            jnp.float32(1.0), jnp.float32(0.0))

        def chunk_body(c, carry):
            base = row0 * HIST + c * chunk_idx
            pltpu.sync_copy(idx_hbm.at[pl.ds(base, chunk_idx)], idx_v)
            pltpu.async_copy(s_hbm.at[idx_v], vals_v, sem).wait()

            def row_body(r, carry2):
                rb = r * HIST
                acc = vals_v[pl.ds(rb + tail_at, _LANES)] * tail_mask
                for j in range(n_full):
                    acc = acc + vals_v[pl.ds(rb + j * _LANES, _LANES)]
                part_v[pl.ds((c * chunk_rows + r) * _LANES, _LANES)] = acc
                return carry2

            lax.fori_loop(0, chunk_rows, row_body, 0)
            return carry

        lax.fori_loop(0, n_chunks, chunk_body, 0)
        pltpu.sync_copy(part_v, out_hbm.at[pl.ds(row0 * _LANES, rows_w * _LANES)])

    return sc_kernel


_sc_gather_sum = _make_sc_gather_sum()


def _reduce16_body(p_ref, o_ref):
    o_ref[...] = jnp.sum(p_ref[...], axis=1, keepdims=True)


def _reduce16(part):
    blk = 2048
    return pl.pallas_call(
        _reduce16_body,
        grid=(BATCH // blk,),
        in_specs=[pl.BlockSpec((blk, _LANES), lambda i: (i, 0))],
        out_specs=pl.BlockSpec((blk, 1), lambda i: (i, 0)),
        out_shape=jax.ShapeDtypeStruct((BATCH, 1), jnp.float32),
    )(part)


def kernel(batch_word_idxs, table, weights):
    s = _table_matvec(table, weights).reshape(VOCAB)
    idx_flat = batch_word_idxs.reshape(-1).astype(jnp.int32)
    part = _sc_gather_sum(idx_flat, s).reshape(BATCH, _LANES)
    return _reduce16(part)


# speedup vs baseline: 1.3010x; 1.3010x over previous
import functools

import jax
import jax.numpy as jnp
from jax import lax
from jax.experimental import pallas as pl
from jax.experimental.pallas import tpu as pltpu
from jax.experimental.pallas import tpu_sc as plsc

VOCAB = 1000000
DIM = 100
BATCH = 16384
HIST = 200

_LANES = 16
_ROW_BLK = 25000


def _matvec_body(t_ref, w_ref, o_ref):
    o_ref[...] = (lax.dot_general(
        w_ref[...], t_ref[...],
        (((1,), (1,)), ((), ())),
        preferred_element_type=jnp.float32,
    ) * (1.0 / HIST))[None]


def _table_matvec(table, weights):
    wt = weights.reshape(1, DIM)
    return pl.pallas_call(
        _matvec_body,
        grid=(VOCAB // _ROW_BLK,),
        in_specs=[
            pl.BlockSpec((_ROW_BLK, DIM), lambda i: (i, 0)),
            pl.BlockSpec((1, DIM), lambda i: (0, 0)),
        ],
        out_specs=pl.BlockSpec((1, 1, _ROW_BLK), lambda i: (i, 0, 0)),
        out_shape=jax.ShapeDtypeStruct(
            (VOCAB // _ROW_BLK, 1, _ROW_BLK), jnp.float32),
    )(table, wt)


def _make_sc_gather_sum():
    nc, ns = 2, 16
    nw = nc * ns
    rows_w = BATCH // nw
    chunk_rows = 128
    n_chunks = rows_w // chunk_rows
    chunk_idx = chunk_rows * HIST

    mesh = plsc.VectorSubcoreMesh(core_axis_name="c", subcore_axis_name="s")
    n_full = HIST // _LANES
    tail_at = HIST - _LANES

    @functools.partial(
        pl.kernel,
        mesh=mesh,
        out_type=jax.ShapeDtypeStruct((BATCH * _LANES,), jnp.float32),
        scratch_types=[
            pltpu.VMEM((chunk_idx,), jnp.int32),
            pltpu.VMEM((chunk_idx,), jnp.float32),
            pltpu.VMEM((rows_w * _LANES,), jnp.float32),
            pltpu.SemaphoreType.DMA,
        ],
    )
    def sc_kernel(idx_hbm, s_hbm, out_hbm, idx_v, vals_v, part_v, sem):
        wid = lax.axis_index("s") * nc + lax.axis_index("c")
        row0 = wid * rows_w
        lane = lax.iota(jnp.int32, _LANES)
        tail_mask = jnp.where(
            lane >= (n_full * _LANES - tail_at),
            jnp.float32(1.0), jnp.float32(0.0))

        def chunk_body(c, carry):
            base = row0 * HIST + c * chunk_idx
            pltpu.sync_copy(idx_hbm.at[pl.ds(base, chunk_idx)], idx_v)
            pltpu.async_copy(s_hbm.at[idx_v], vals_v, sem).wait()

            def row_body(r, carry2):
                rb = r * HIST
                acc = vals_v[pl.ds(rb + tail_at, _LANES)] * tail_mask
                for j in range(n_full):
                    acc = acc + vals_v[pl.ds(rb + j * _LANES, _LANES)]
                part_v[pl.ds((c * chunk_rows + r) * _LANES, _LANES)] = acc
                return carry2

            lax.fori_loop(0, chunk_rows, row_body, 0)
            return carry

        lax.fori_loop(0, n_chunks, chunk_body, 0)
        pltpu.sync_copy(part_v, out_hbm.at[pl.ds(row0 * _LANES, rows_w * _LANES)])

    return sc_kernel


_sc_gather_sum = _make_sc_gather_sum()


def _reduce16_body(p_ref, o_ref):
    o_ref[...] = jnp.sum(p_ref[...], axis=1, keepdims=True)


def _reduce16(part):
    blk = 2048
    return pl.pallas_call(
        _reduce16_body,
        grid=(BATCH // blk,),
        in_specs=[pl.BlockSpec((blk, _LANES), lambda i: (i, 0))],
        out_specs=pl.BlockSpec((blk, 1), lambda i: (i, 0)),
        out_shape=jax.ShapeDtypeStruct((BATCH, 1), jnp.float32),
    )(part)


def kernel(batch_word_idxs, table, weights):
    s = _table_matvec(table, weights).reshape(VOCAB)
    idx_flat = batch_word_idxs.reshape(-1).astype(jnp.int32)
    part = _sc_gather_sum(idx_flat, s).reshape(BATCH, _LANES)
    return _reduce16(part)
